# baseline (speedup 1.0000x reference)
import jax
import jax.numpy as jnp
from jax.experimental import pallas as pl

NUM_FEATURES = 128


def _id_body(x_ref, o_ref):
    o_ref[...] = x_ref[...]


def kernel(x_, edge_index, y, params):
    relu = jax.nn.relu
    N = x_.shape[0]
    x_ = pl.pallas_call(
        _id_body, out_shape=jax.ShapeDtypeStruct(x_.shape, x_.dtype)
    )(x_)
    D_inv = (1.0 / (x_[:, NUM_FEATURES] + 1.0)).reshape(-1, 1)
    graph_fea = x_[:, NUM_FEATURES + 1:]
    src, dst = edge_index[0], edge_index[1]
    deg = jnp.bincount(dst, length=N).astype(jnp.float32) + 1.0
    dn = jax.lax.rsqrt(deg)
    norm = (dn[src] * dn[dst])[:, None]

    def conv(h_in, name):
        W, b = params[name]
        h = h_in @ W + b
        agg = jnp.zeros_like(h).at[dst].add(h[src] * norm)
        return agg, h

    def lin(h, name):
        W, b = params[name]
        return h @ W + b

    x1a, x1b = conv(x_, 'conv1')
    x2a, x2b = conv(x_, 'conv4')
    x1 = relu(x1a + x2b * D_inv)
    x2 = relu(x2a + x1b * D_inv)
    z1a, z1b = conv(x1, 'conv2')
    z2a, z2b = conv(x2, 'conv5')
    z1 = relu(z1a + z2b * D_inv)
    z2 = relu(z2a + z1b * D_inv)
    v1a, v1b = conv(z1, 'conv3')
    v2a, v2b = conv(z2, 'conv6')
    v1 = relu(v1a + v2b * D_inv)
    v2 = relu(v2a + v1b * D_inv)
    q = jnp.concatenate([x1, z1, v1], axis=1)
    p = jnp.concatenate([x2, z2, v2], axis=1)
    node_fea = jnp.concatenate([q, p], axis=1)

    a1 = relu(lin(node_fea, 'fl1'))
    a2 = relu(lin(a1, 'fl2'))
    a3 = relu(lin(a2, 'fl3'))
    b1 = relu(lin(graph_fea, 'fl4'))
    b2 = relu(lin(b1, 'fl5'))
    b3 = relu(lin(b2, 'fl6'))
    fea = jnp.concatenate([a1, a2, a3, b1, b2, b3], axis=1)

    def classify(h, n1, n2, n3):
        h1 = relu(lin(h, n1))
        h2 = relu(lin(h1, n2))
        logits = lin(h2, n3)
        logp = jax.nn.log_softmax(logits, axis=-1)
        loss = -jnp.mean(logp[jnp.arange(logp.shape[0]), y])
        pred = jnp.argmax(logp, axis=1)
        return loss, pred, logits

    lossO, predO, logitsO = classify(fea, 'co1', 'co2', 'co3')
    loss1, pred1, logits1 = classify(node_fea, 'c11', 'c12', 'c13')
    return lossO + loss1, predO, logitsO


# speedup vs baseline: 5.1273x; 5.1273x over previous
import functools

import jax
import jax.numpy as jnp
from jax import lax
from jax.experimental import pallas as pl
from jax.experimental.pallas import tpu as pltpu
from jax.experimental.pallas import tpu_sc as plsc

N = 10000
E = 320000
NFEAT = 135
NF = 128
NHID = 128
NCLS = 7

NP = 10240
NC, NS, L = 2, 16, 16
CH = 128
RPS = 160
EROWS = NS * RPS
EPAD = EROWS * CH
EPW = EPAD // (NC * NS)
ROWS_N = NP // NS
NBUF = 4

BR = 512
NPROG = NP // BR


_DROWS = EROWS // (NC * NS)


def _deg_body(dst3, ones_hbm, zeros_hbm, out, dstv, ones_v, acc):
    c = lax.axis_index("c")
    s = lax.axis_index("s")
    row0 = (c * NS + s) * _DROWS
    nb = s * ROWS_N
    pltpu.sync_copy(dst3.at[pl.ds(row0, _DROWS)], dstv)
    pltpu.sync_copy(ones_hbm, ones_v)
    pltpu.sync_copy(zeros_hbm.at[pl.ds(nb, ROWS_N)], acc.at[pl.ds(nb, ROWS_N)])
    plsc.subcore_barrier()

    def body(j, _):
        pltpu.sync_copy(ones_v, acc.at[dstv.at[j]], add=True)
        return 0

    lax.fori_loop(0, _DROWS, body, 0)
    plsc.subcore_barrier()
    pltpu.sync_copy(acc.at[pl.ds(nb, ROWS_N)], out.at[pl.ds(c * NP + nb, ROWS_N)])


GRP = 8
NG = RPS // GRP


def _agg_body(edges3, h0, h1, zeros_hbm, out0, out1, ibuf, gbuf, acc):
    c = lax.axis_index("c")
    s = lax.axis_index("s")
    row0 = s * RPS
    nb = s * ROWS_N
    pltpu.sync_copy(zeros_hbm.at[pl.ds(nb, ROWS_N)], acc.at[pl.ds(nb, ROWS_N)])
    plsc.subcore_barrier()

    def run(h, out):
        def body(g, _):
            pltpu.sync_copy(edges3.at[pl.ds(row0 + g * GRP, GRP)], ibuf)
            for k in range(GRP):
                pltpu.sync_copy(h.at[ibuf.at[k, 0]], gbuf)
                pltpu.sync_copy(gbuf, acc.at[ibuf.at[k, 1]], add=True)
            return 0

        lax.fori_loop(0, NG, body, 0)
        plsc.subcore_barrier()
        pltpu.sync_copy(acc.at[pl.ds(nb, ROWS_N)], out.at[pl.ds(nb, ROWS_N)])

    @pl.when(c == 0)
    def _():
        run(h0, out0)

    @pl.when(c == 1)
    def _():
        run(h1, out1)


@functools.cache
def _get_sc_kernels():
    mesh = plsc.VectorSubcoreMesh(core_axis_name="c", subcore_axis_name="s",
                                  num_cores=NC, num_subcores=NS)
    deg = pl.kernel(
        _deg_body,
        out_type=jax.ShapeDtypeStruct((NC * NP, NHID), jnp.float32),
        mesh=mesh,
        scratch_types=[
            pltpu.VMEM((_DROWS, CH), jnp.int32),
            pltpu.VMEM((CH, NHID), jnp.float32),
            pltpu.VMEM_SHARED((NP, NHID), jnp.float32),
        ],
    )
    agg = pl.kernel(
        _agg_body,
        out_type=[jax.ShapeDtypeStruct((NP, NHID), jnp.float32)] * 2,
        mesh=mesh,
        scratch_types=[
            pltpu.VMEM((GRP, 2, CH), jnp.int32),
            pltpu.VMEM((CH, NHID), jnp.float32),
            pltpu.VMEM_SHARED((NP, NHID), jnp.float32),
        ],
    )
    return deg, agg


def _dot(a, b):
    return jnp.dot(a, b, preferred_element_type=jnp.float32)


def _lin1_body(x_ref, w1_ref, b1_ref, w4_ref, b4_ref,
               h1_ref, h2_ref, dinv_ref):
    x = x_ref[...]
    h1_ref[...] = _dot(x, w1_ref[...]) + b1_ref[...]
    h2_ref[...] = _dot(x, w4_ref[...]) + b4_ref[...]
    dinv_ref[...] = 1.0 / (x[:, NF:NF + 1] + 1.0)


def _scale_body(p0_ref, p1_ref, h1_ref, h2_ref, dn_ref, h1s_ref, h2s_ref):
    d = p0_ref[:, 0:1] + p1_ref[:, 0:1] + 1.0
    dn = lax.rsqrt(d)
    dn_ref[...] = dn
    h1s_ref[...] = h1_ref[...] * dn
    h2s_ref[...] = h2_ref[...] * dn


def _mid_body(a0_ref, a1_ref, hp1_ref, hp2_ref, dn_ref, dinv_ref,
              wa_ref, ba_ref, wb_ref, bb_ref,
              u1_ref, u2_ref, g1_ref, g2_ref, g1s_ref, g2s_ref):
    dn = dn_ref[...]
    dinv = dinv_ref[...]
    u1 = jnp.maximum(dn * a0_ref[...] + hp2_ref[...] * dinv, 0.0)
    u2 = jnp.maximum(dn * a1_ref[...] + hp1_ref[...] * dinv, 0.0)
    g1 = _dot(u1, wa_ref[...]) + ba_ref[...]
    g2 = _dot(u2, wb_ref[...]) + bb_ref[...]
    u1_ref[...] = u1
    u2_ref[...] = u2
    g1_ref[...] = g1
    g2_ref[...] = g2
    g1s_ref[...] = g1 * dn
    g2s_ref[...] = g2 * dn


def _log_softmax(l):
    m = jnp.max(l, axis=1, keepdims=True)
    e = jnp.exp(l - m)
    return l - m - jnp.log(jnp.sum(e, axis=1, keepdims=True))


def _final_body(a0_ref, a1_ref, h5_ref, h6_ref, dn_ref, dinv_ref,
                x1_ref, x2_ref, z1_ref, z2_ref, gf_ref, y_ref,
                fl1w_ref, fl1b_ref, fl2w_ref, fl2b_ref, fl3w_ref, fl3b_ref,
                fl4w_ref, fl4b_ref, fl5w_ref, fl5b_ref, fl6w_ref, fl6b_ref,
                co1w_ref, co1b_ref, co2w_ref, co2b_ref, co3w_ref, co3b_ref,
                c11w_ref, c11b_ref, c12w_ref, c12b_ref, c13w_ref, c13b_ref,
                loss_ref, pred_ref, logits_ref, acc_ref):
    pid = pl.program_id(0)
    dn = dn_ref[...]
    dinv = dinv_ref[...]
    v1 = jnp.maximum(dn * a0_ref[...] + h6_ref[...] * dinv, 0.0)
    v2 = jnp.maximum(dn * a1_ref[...] + h5_ref[...] * dinv, 0.0)
    x1, x2, z1, z2 = x1_ref[...], x2_ref[...], z1_ref[...], z2_ref[...]

    def cat6(w_ref, parts, sizes):
        o = 0
        acc = None
        for p, sz in zip(parts, sizes):
            term = _dot(p, w_ref[pl.ds(o, sz), :])
            acc = term if acc is None else acc + term
            o += sz
        return acc

    nf_parts = (x1, z1, v1, x2, z2, v2)
    nf_sizes = (128, 128, 128, 128, 128, 128)
    a1 = jnp.maximum(cat6(fl1w_ref, nf_parts, nf_sizes) + fl1b_ref[...], 0.0)
    a2 = jnp.maximum(_dot(a1, fl2w_ref[...]) + fl2b_ref[...], 0.0)
    a3 = jnp.maximum(_dot(a2, fl3w_ref[...]) + fl3b_ref[...], 0.0)
    gf = gf_ref[...]
    b1 = jnp.maximum(_dot(gf, fl4w_ref[...]) + fl4b_ref[...], 0.0)
    b2 = jnp.maximum(_dot(b1, fl5w_ref[...]) + fl5b_ref[...], 0.0)
    b3 = jnp.maximum(_dot(b2, fl6w_ref[...]) + fl6b_ref[...], 0.0)

    fea_parts = (a1, a2, a3, b1, b2, b3)
    fea_sizes = (256, 128, 128, 64, 32, 32)
    hO = jnp.maximum(cat6(co1w_ref, fea_parts, fea_sizes) + co1b_ref[...], 0.0)
    hO2 = jnp.maximum(_dot(hO, co2w_ref[...]) + co2b_ref[...], 0.0)
    logitsO = _dot(hO2, co3w_ref[...]) + co3b_ref[...]

    h1c = jnp.maximum(cat6(c11w_ref, nf_parts, nf_sizes) + c11b_ref[...], 0.0)
    h2c = jnp.maximum(_dot(h1c, c12w_ref[...]) + c12b_ref[...], 0.0)
    logits1 = _dot(h2c, c13w_ref[...]) + c13b_ref[...]

    lpO = _log_softmax(logitsO)
    lp1 = _log_softmax(logits1)
    cls = lax.broadcasted_iota(jnp.int32, (BR, NCLS), 1)
    onehot = (cls == y_ref[...]).astype(jnp.float32)
    rid = pid * BR + lax.broadcasted_iota(jnp.int32, (BR, 1), 0)
    mask = (rid < N).astype(jnp.float32)
    partO = jnp.sum(jnp.sum(lpO * onehot, axis=1, keepdims=True) * mask)
    part1 = jnp.sum(jnp.sum(lp1 * onehot, axis=1, keepdims=True) * mask)

    @pl.when(pid == 0)
    def _():
        acc_ref[0] = 0.0
        acc_ref[1] = 0.0

    acc_ref[0] += partO
    acc_ref[1] += part1

    @pl.when(pid == NPROG - 1)
    def _():
        loss_ref[...] = jnp.full((1, 1), -(acc_ref[0] + acc_ref[1]) / N,
                                 jnp.float32)

    m = logitsO[:, 0:1]
    idx = jnp.zeros((BR, 1), jnp.int32)
    for cc in range(1, NCLS):
        col = logitsO[:, cc:cc + 1]
        gt = col > m
        idx = jnp.where(gt, cc, idx)
        m = jnp.where(gt, col, m)
    pred_ref[...] = idx
    logits_ref[...] = logitsO


def _row_spec(cols):
    return pl.BlockSpec((BR, cols), lambda i: (i, 0))


def _full_spec(shape):
    nd = len(shape)
    return pl.BlockSpec(shape, lambda i, _n=nd: (0,) * nd)


def _lin1(xp, w1, b1, w4, b4):
    return pl.pallas_call(
        _lin1_body,
        grid=(NPROG,),
        in_specs=[
            _row_spec(NFEAT),
            _full_spec(w1.shape), _full_spec(b1.shape),
            _full_spec(w4.shape), _full_spec(b4.shape),
        ],
        out_specs=[_row_spec(NHID), _row_spec(NHID), _row_spec(1)],
        out_shape=[
            jax.ShapeDtypeStruct((NP, NHID), jnp.float32),
            jax.ShapeDtypeStruct((NP, NHID), jnp.float32),
            jax.ShapeDtypeStruct((NP, 1), jnp.float32),
        ],
    )(xp, w1, b1, w4, b4)


_BRS = 2048


def _scale(p0, p1, h1, h2):
    return pl.pallas_call(
        _scale_body,
        grid=(NP // _BRS,),
        in_specs=[
            pl.BlockSpec((_BRS, NHID), lambda i: (i, 0)),
            pl.BlockSpec((_BRS, NHID), lambda i: (i, 0)),
            pl.BlockSpec((_BRS, NHID), lambda i: (i, 0)),
            pl.BlockSpec((_BRS, NHID), lambda i: (i, 0)),
        ],
        out_specs=[
            pl.BlockSpec((_BRS, 1), lambda i: (i, 0)),
            pl.BlockSpec((_BRS, NHID), lambda i: (i, 0)),
            pl.BlockSpec((_BRS, NHID), lambda i: (i, 0)),
        ],
        out_shape=[
            jax.ShapeDtypeStruct((NP, 1), jnp.float32),
            jax.ShapeDtypeStruct((NP, NHID), jnp.float32),
            jax.ShapeDtypeStruct((NP, NHID), jnp.float32),
        ],
    )(p0, p1, h1, h2)


def _mid(a0, a1, hp1, hp2, dn, dinv, wa, ba, wb, bb):
    return pl.pallas_call(
        _mid_body,
        grid=(NPROG,),
        in_specs=[
            _row_spec(NHID), _row_spec(NHID),
            _row_spec(NHID), _row_spec(NHID),
            _row_spec(1), _row_spec(1),
            _full_spec(wa.shape), _full_spec(ba.shape),
            _full_spec(wb.shape), _full_spec(bb.shape),
        ],
        out_specs=[_row_spec(NHID)] * 6,
        out_shape=[jax.ShapeDtypeStruct((NP, NHID), jnp.float32)] * 6,
    )(a0, a1, hp1, hp2, dn, dinv, wa, ba, wb, bb)


def _final(a0, a1, h5, h6, dn, dinv, x1, x2, z1, z2, gf, y2, wlist):
    in_specs = [
        _row_spec(NHID), _row_spec(NHID),
        _row_spec(NHID), _row_spec(NHID),
        _row_spec(1), _row_spec(1),
        _row_spec(NHID), _row_spec(NHID),
        _row_spec(NHID), _row_spec(NHID),
        _row_spec(6), _row_spec(1),
    ] + [_full_spec(w.shape) for w in wlist]
    return pl.pallas_call(
        _final_body,
        grid=(NPROG,),
        in_specs=in_specs,
        out_specs=[
            pl.BlockSpec((1, 1), lambda i: (0, 0)),
            _row_spec(1),
            _row_spec(NCLS),
        ],
        out_shape=[
            jax.ShapeDtypeStruct((1, 1), jnp.float32),
            jax.ShapeDtypeStruct((NP, 1), jnp.int32),
            jax.ShapeDtypeStruct((NP, NCLS), jnp.float32),
        ],
        scratch_shapes=[pltpu.SMEM((2,), jnp.float32)],
    )(a0, a1, h5, h6, dn, dinv, x1, x2, z1, z2, gf, y2, *wlist)


def kernel(x_, edge_index, y, params):
    f32 = jnp.float32
    xp = jnp.pad(x_, ((0, NP - N), (0, 0)))
    gf = xp[:, NF + 1:]
    y2 = jnp.pad(y, (0, NP - N)).reshape(NP, 1)

    src = edge_index[0]
    dst = edge_index[1]
    epad = jnp.full((EPAD - E,), NP - 1, jnp.int32)
    src3 = jnp.concatenate([src, epad]).reshape(EROWS, CH)
    dst3 = jnp.concatenate([dst, epad]).reshape(EROWS, CH)
    edges3 = jnp.stack([src3, dst3], axis=1)
    zeros128 = jnp.zeros((NP, NHID), f32)

    def wb(name):
        W, b = params[name]
        return W, b.reshape(1, -1)

    w1, b1 = wb('conv1')
    w4, b4 = wb('conv4')

    _deg_kernel, _agg_kernel = _get_sc_kernels()
    ones128 = jnp.ones((CH, NHID), f32)
    deg_out = _deg_kernel(dst3, ones128, zeros128)
    p0, p1 = deg_out[:NP], deg_out[NP:]
    h1, h2, dinv = _lin1(xp, w1, b1, w4, b4)
    dn, h1s, h2s = _scale(p0, p1, h1, h2)

    w2, b2 = wb('conv2')
    w5, b5 = wb('conv5')
    a0, a1 = _agg_kernel(edges3, h1s, h2s, zeros128)
    x1, x2, h3, h4, h3s, h4s = _mid(a0, a1, h1, h2, dn, dinv, w2, b2, w5, b5)

    w3, b3 = wb('conv3')
    w6, b6 = wb('conv6')
    a0, a1 = _agg_kernel(edges3, h3s, h4s, zeros128)
    z1, z2, h5, h6, h5s, h6s = _mid(a0, a1, h3, h4, dn, dinv, w3, b3, w6, b6)

    a0, a1 = _agg_kernel(edges3, h5s, h6s, zeros128)

    wlist = []
    for name in ('fl1', 'fl2', 'fl3', 'fl4', 'fl5', 'fl6',
                 'co1', 'co2', 'co3', 'c11', 'c12', 'c13'):
        W, b = wb(name)
        wlist += [W, b]
    loss, pred, logits = _final(a0, a1, h5, h6, dn, dinv,
                                x1, x2, z1, z2, gf, y2, wlist)
    return loss.reshape(()), pred[:N, 0], logits[:N]


# speedup vs baseline: 5.9911x; 1.1685x over previous
import functools

import jax
import jax.numpy as jnp
from jax import lax
from jax.experimental import pallas as pl
from jax.experimental.pallas import tpu as pltpu
from jax.experimental.pallas import tpu_sc as plsc

N = 10000
E = 320000
NFEAT = 135
NF = 128
NHID = 128
NCLS = 7

NP = 10240
NC, NS, L = 2, 16, 16
CH = 128
RPS = 160
EROWS = NS * RPS
EPAD = EROWS * CH
EPW = EPAD // (NC * NS)
ROWS_N = NP // NS
NBUF = 4

BR = 512
NPROG = NP // BR


_DROWS = EROWS // (NC * NS)


def _deg_body(dst3, ones_hbm, zeros_hbm, out, dstv, ones_v, acc):
    c = lax.axis_index("c")
    s = lax.axis_index("s")
    row0 = (c * NS + s) * _DROWS
    nb = s * ROWS_N
    pltpu.sync_copy(dst3.at[pl.ds(row0, _DROWS)], dstv)
    pltpu.sync_copy(ones_hbm, ones_v)
    pltpu.sync_copy(zeros_hbm.at[pl.ds(nb, ROWS_N)], acc.at[pl.ds(nb, ROWS_N)])
    plsc.subcore_barrier()

    def body(j, _):
        pltpu.sync_copy(ones_v, acc.at[dstv.at[j]], add=True)
        return 0

    lax.fori_loop(0, _DROWS, body, 0)
    plsc.subcore_barrier()
    pltpu.sync_copy(acc.at[pl.ds(nb, ROWS_N)], out.at[pl.ds(c * NP + nb, ROWS_N)])


GRP = 8
NG = RPS // GRP


def _agg_body(edges3, h0, h1, zeros_hbm, out0, out1, ibuf, gbuf, acc,
              isem, gsem, ssem):
    c = lax.axis_index("c")
    s = lax.axis_index("s")
    row0 = s * RPS
    nb = s * ROWS_N
    pltpu.sync_copy(zeros_hbm.at[pl.ds(nb, ROWS_N)], acc.at[pl.ds(nb, ROWS_N)])
    plsc.subcore_barrier()

    def run(h, out):
        def icopy(g):
            pltpu.async_copy(edges3.at[pl.ds(row0 + (g % NG) * GRP, GRP)],
                             ibuf.at[g % 2], isem)

        def iwait(g):
            pltpu.make_async_copy(edges3.at[pl.ds(row0, GRP)],
                                  ibuf.at[g % 2], isem).wait()

        def gstart(gm, k):
            pltpu.async_copy(h.at[ibuf.at[gm, k, 0]],
                             gbuf.at[k % 2], gsem.at[k % 2])

        def gwait(gm, k):
            pltpu.make_async_copy(h.at[ibuf.at[gm, k, 0]],
                                  gbuf.at[k % 2], gsem.at[k % 2]).wait()

        def sstart(gm, k):
            pltpu.async_copy(gbuf.at[k % 2], acc.at[ibuf.at[gm, k, 1]],
                             ssem.at[k % 2], add=True)

        def swait(gm, k):
            pltpu.make_async_copy(gbuf.at[k % 2], acc.at[ibuf.at[gm, k, 1]],
                                  ssem.at[k % 2]).wait()

        icopy(0)

        def body(g, _):
            gm = g % 2
            iwait(g)
            icopy(g + 1)
            gstart(gm, 0)
            gstart(gm, 1)
            for k in range(GRP):
                gwait(gm, k)
                sstart(gm, k)
                if k + 2 < GRP:
                    swait(gm, k)
                    gstart(gm, k + 2)
            swait(gm, GRP - 2)
            swait(gm, GRP - 1)
            return 0

        lax.fori_loop(0, NG, body, 0)
        iwait(NG)
        plsc.subcore_barrier()
        pltpu.sync_copy(acc.at[pl.ds(nb, ROWS_N)], out.at[pl.ds(nb, ROWS_N)])

    @pl.when(c == 0)
    def _():
        run(h0, out0)

    @pl.when(c == 1)
    def _():
        run(h1, out1)


@functools.cache
def _get_sc_kernels():
    mesh = plsc.VectorSubcoreMesh(core_axis_name="c", subcore_axis_name="s",
                                  num_cores=NC, num_subcores=NS)
    deg = pl.kernel(
        _deg_body,
        out_type=jax.ShapeDtypeStruct((NC * NP, NHID), jnp.float32),
        mesh=mesh,
        scratch_types=[
            pltpu.VMEM((_DROWS, CH), jnp.int32),
            pltpu.VMEM((CH, NHID), jnp.float32),
            pltpu.VMEM_SHARED((NP, NHID), jnp.float32),
        ],
    )
    agg = pl.kernel(
        _agg_body,
        out_type=[jax.ShapeDtypeStruct((NP, NHID), jnp.float32)] * 2,
        mesh=mesh,
        scratch_types=[
            pltpu.VMEM((2, GRP, 2, CH), jnp.int32),
            pltpu.VMEM((2, CH, NHID), jnp.float32),
            pltpu.VMEM_SHARED((NP, NHID), jnp.float32),
            pltpu.SemaphoreType.DMA,
            pltpu.SemaphoreType.DMA((2,)),
            pltpu.SemaphoreType.DMA((2,)),
        ],
    )
    return deg, agg


def _dot(a, b):
    return jnp.dot(a, b, preferred_element_type=jnp.float32)


def _lin1_body(x_ref, w1_ref, b1_ref, w4_ref, b4_ref,
               h1_ref, h2_ref, dinv_ref):
    x = x_ref[...]
    h1_ref[...] = _dot(x, w1_ref[...]) + b1_ref[...]
    h2_ref[...] = _dot(x, w4_ref[...]) + b4_ref[...]
    dinv_ref[...] = 1.0 / (x[:, NF:NF + 1] + 1.0)


def _scale_body(p0_ref, p1_ref, h1_ref, h2_ref, dn_ref, h1s_ref, h2s_ref):
    d = p0_ref[:, 0:1] + p1_ref[:, 0:1] + 1.0
    dn = lax.rsqrt(d)
    dn_ref[...] = dn
    h1s_ref[...] = h1_ref[...] * dn
    h2s_ref[...] = h2_ref[...] * dn


def _mid_body(a0_ref, a1_ref, hp1_ref, hp2_ref, dn_ref, dinv_ref,
              wa_ref, ba_ref, wb_ref, bb_ref,
              u1_ref, u2_ref, g1_ref, g2_ref, g1s_ref, g2s_ref):
    dn = dn_ref[...]
    dinv = dinv_ref[...]
    u1 = jnp.maximum(dn * a0_ref[...] + hp2_ref[...] * dinv, 0.0)
    u2 = jnp.maximum(dn * a1_ref[...] + hp1_ref[...] * dinv, 0.0)
    g1 = _dot(u1, wa_ref[...]) + ba_ref[...]
    g2 = _dot(u2, wb_ref[...]) + bb_ref[...]
    u1_ref[...] = u1
    u2_ref[...] = u2
    g1_ref[...] = g1
    g2_ref[...] = g2
    g1s_ref[...] = g1 * dn
    g2s_ref[...] = g2 * dn


def _log_softmax(l):
    m = jnp.max(l, axis=1, keepdims=True)
    e = jnp.exp(l - m)
    return l - m - jnp.log(jnp.sum(e, axis=1, keepdims=True))


def _final_body(a0_ref, a1_ref, h5_ref, h6_ref, dn_ref, dinv_ref,
                x1_ref, x2_ref, z1_ref, z2_ref, gf_ref, y_ref,
                fl1w_ref, fl1b_ref, fl2w_ref, fl2b_ref, fl3w_ref, fl3b_ref,
                fl4w_ref, fl4b_ref, fl5w_ref, fl5b_ref, fl6w_ref, fl6b_ref,
                co1w_ref, co1b_ref, co2w_ref, co2b_ref, co3w_ref, co3b_ref,
                c11w_ref, c11b_ref, c12w_ref, c12b_ref, c13w_ref, c13b_ref,
                loss_ref, pred_ref, logits_ref, acc_ref):
    pid = pl.program_id(0)
    dn = dn_ref[...]
    dinv = dinv_ref[...]
    v1 = jnp.maximum(dn * a0_ref[...] + h6_ref[...] * dinv, 0.0)
    v2 = jnp.maximum(dn * a1_ref[...] + h5_ref[...] * dinv, 0.0)
    x1, x2, z1, z2 = x1_ref[...], x2_ref[...], z1_ref[...], z2_ref[...]

    def cat6(w_ref, parts, sizes):
        o = 0
        acc = None
        for p, sz in zip(parts, sizes):
            term = _dot(p, w_ref[pl.ds(o, sz), :])
            acc = term if acc is None else acc + term
            o += sz
        return acc

    nf_parts = (x1, z1, v1, x2, z2, v2)
    nf_sizes = (128, 128, 128, 128, 128, 128)
    a1 = jnp.maximum(cat6(fl1w_ref, nf_parts, nf_sizes) + fl1b_ref[...], 0.0)
    a2 = jnp.maximum(_dot(a1, fl2w_ref[...]) + fl2b_ref[...], 0.0)
    a3 = jnp.maximum(_dot(a2, fl3w_ref[...]) + fl3b_ref[...], 0.0)
    gf = gf_ref[...]
    b1 = jnp.maximum(_dot(gf, fl4w_ref[...]) + fl4b_ref[...], 0.0)
    b2 = jnp.maximum(_dot(b1, fl5w_ref[...]) + fl5b_ref[...], 0.0)
    b3 = jnp.maximum(_dot(b2, fl6w_ref[...]) + fl6b_ref[...], 0.0)

    fea_parts = (a1, a2, a3, b1, b2, b3)
    fea_sizes = (256, 128, 128, 64, 32, 32)
    hO = jnp.maximum(cat6(co1w_ref, fea_parts, fea_sizes) + co1b_ref[...], 0.0)
    hO2 = jnp.maximum(_dot(hO, co2w_ref[...]) + co2b_ref[...], 0.0)
    logitsO = _dot(hO2, co3w_ref[...]) + co3b_ref[...]

    h1c = jnp.maximum(cat6(c11w_ref, nf_parts, nf_sizes) + c11b_ref[...], 0.0)
    h2c = jnp.maximum(_dot(h1c, c12w_ref[...]) + c12b_ref[...], 0.0)
    logits1 = _dot(h2c, c13w_ref[...]) + c13b_ref[...]

    lpO = _log_softmax(logitsO)
    lp1 = _log_softmax(logits1)
    cls = lax.broadcasted_iota(jnp.int32, (BR, NCLS), 1)
    onehot = (cls == y_ref[...]).astype(jnp.float32)
    rid = pid * BR + lax.broadcasted_iota(jnp.int32, (BR, 1), 0)
    mask = (rid < N).astype(jnp.float32)
    partO = jnp.sum(jnp.sum(lpO * onehot, axis=1, keepdims=True) * mask)
    part1 = jnp.sum(jnp.sum(lp1 * onehot, axis=1, keepdims=True) * mask)

    @pl.when(pid == 0)
    def _():
        acc_ref[0] = 0.0
        acc_ref[1] = 0.0

    acc_ref[0] += partO
    acc_ref[1] += part1

    @pl.when(pid == NPROG - 1)
    def _():
        loss_ref[...] = jnp.full((1, 1), -(acc_ref[0] + acc_ref[1]) / N,
                                 jnp.float32)

    m = logitsO[:, 0:1]
    idx = jnp.zeros((BR, 1), jnp.int32)
    for cc in range(1, NCLS):
        col = logitsO[:, cc:cc + 1]
        gt = col > m
        idx = jnp.where(gt, cc, idx)
        m = jnp.where(gt, col, m)
    pred_ref[...] = idx
    logits_ref[...] = logitsO


def _row_spec(cols):
    return pl.BlockSpec((BR, cols), lambda i: (i, 0))


def _full_spec(shape):
    nd = len(shape)
    return pl.BlockSpec(shape, lambda i, _n=nd: (0,) * nd)


def _lin1(xp, w1, b1, w4, b4):
    return pl.pallas_call(
        _lin1_body,
        grid=(NPROG,),
        in_specs=[
            _row_spec(NFEAT),
            _full_spec(w1.shape), _full_spec(b1.shape),
            _full_spec(w4.shape), _full_spec(b4.shape),
        ],
        out_specs=[_row_spec(NHID), _row_spec(NHID), _row_spec(1)],
        out_shape=[
            jax.ShapeDtypeStruct((NP, NHID), jnp.float32),
            jax.ShapeDtypeStruct((NP, NHID), jnp.float32),
            jax.ShapeDtypeStruct((NP, 1), jnp.float32),
        ],
    )(xp, w1, b1, w4, b4)


_BRS = 2048


def _scale(p0, p1, h1, h2):
    return pl.pallas_call(
        _scale_body,
        grid=(NP // _BRS,),
        in_specs=[
            pl.BlockSpec((_BRS, NHID), lambda i: (i, 0)),
            pl.BlockSpec((_BRS, NHID), lambda i: (i, 0)),
            pl.BlockSpec((_BRS, NHID), lambda i: (i, 0)),
            pl.BlockSpec((_BRS, NHID), lambda i: (i, 0)),
        ],
        out_specs=[
            pl.BlockSpec((_BRS, 1), lambda i: (i, 0)),
            pl.BlockSpec((_BRS, NHID), lambda i: (i, 0)),
            pl.BlockSpec((_BRS, NHID), lambda i: (i, 0)),
        ],
        out_shape=[
            jax.ShapeDtypeStruct((NP, 1), jnp.float32),
            jax.ShapeDtypeStruct((NP, NHID), jnp.float32),
            jax.ShapeDtypeStruct((NP, NHID), jnp.float32),
        ],
    )(p0, p1, h1, h2)


def _mid(a0, a1, hp1, hp2, dn, dinv, wa, ba, wb, bb):
    return pl.pallas_call(
        _mid_body,
        grid=(NPROG,),
        in_specs=[
            _row_spec(NHID), _row_spec(NHID),
            _row_spec(NHID), _row_spec(NHID),
            _row_spec(1), _row_spec(1),
            _full_spec(wa.shape), _full_spec(ba.shape),
            _full_spec(wb.shape), _full_spec(bb.shape),
        ],
        out_specs=[_row_spec(NHID)] * 6,
        out_shape=[jax.ShapeDtypeStruct((NP, NHID), jnp.float32)] * 6,
    )(a0, a1, hp1, hp2, dn, dinv, wa, ba, wb, bb)


def _final(a0, a1, h5, h6, dn, dinv, x1, x2, z1, z2, gf, y2, wlist):
    in_specs = [
        _row_spec(NHID), _row_spec(NHID),
        _row_spec(NHID), _row_spec(NHID),
        _row_spec(1), _row_spec(1),
        _row_spec(NHID), _row_spec(NHID),
        _row_spec(NHID), _row_spec(NHID),
        _row_spec(6), _row_spec(1),
    ] + [_full_spec(w.shape) for w in wlist]
    return pl.pallas_call(
        _final_body,
        grid=(NPROG,),
        in_specs=in_specs,
        out_specs=[
            pl.BlockSpec((1, 1), lambda i: (0, 0)),
            _row_spec(1),
            _row_spec(NCLS),
        ],
        out_shape=[
            jax.ShapeDtypeStruct((1, 1), jnp.float32),
            jax.ShapeDtypeStruct((NP, 1), jnp.int32),
            jax.ShapeDtypeStruct((NP, NCLS), jnp.float32),
        ],
        scratch_shapes=[pltpu.SMEM((2,), jnp.float32)],
    )(a0, a1, h5, h6, dn, dinv, x1, x2, z1, z2, gf, y2, *wlist)


def kernel(x_, edge_index, y, params):
    f32 = jnp.float32
    xp = jnp.pad(x_, ((0, NP - N), (0, 0)))
    gf = xp[:, NF + 1:]
    y2 = jnp.pad(y, (0, NP - N)).reshape(NP, 1)

    src = edge_index[0]
    dst = edge_index[1]
    epad = jnp.full((EPAD - E,), NP - 1, jnp.int32)
    src3 = jnp.concatenate([src, epad]).reshape(EROWS, CH)
    dst3 = jnp.concatenate([dst, epad]).reshape(EROWS, CH)
    edges3 = jnp.stack([src3, dst3], axis=1)
    zeros128 = jnp.zeros((NP, NHID), f32)

    def wb(name):
        W, b = params[name]
        return W, b.reshape(1, -1)

    w1, b1 = wb('conv1')
    w4, b4 = wb('conv4')

    _deg_kernel, _agg_kernel = _get_sc_kernels()
    ones128 = jnp.ones((CH, NHID), f32)
    deg_out = _deg_kernel(dst3, ones128, zeros128)
    p0, p1 = deg_out[:NP], deg_out[NP:]
    h1, h2, dinv = _lin1(xp, w1, b1, w4, b4)
    dn, h1s, h2s = _scale(p0, p1, h1, h2)

    w2, b2 = wb('conv2')
    w5, b5 = wb('conv5')
    a0, a1 = _agg_kernel(edges3, h1s, h2s, zeros128)
    x1, x2, h3, h4, h3s, h4s = _mid(a0, a1, h1, h2, dn, dinv, w2, b2, w5, b5)

    w3, b3 = wb('conv3')
    w6, b6 = wb('conv6')
    a0, a1 = _agg_kernel(edges3, h3s, h4s, zeros128)
    z1, z2, h5, h6, h5s, h6s = _mid(a0, a1, h3, h4, dn, dinv, w3, b3, w6, b6)

    a0, a1 = _agg_kernel(edges3, h5s, h6s, zeros128)

    wlist = []
    for name in ('fl1', 'fl2', 'fl3', 'fl4', 'fl5', 'fl6',
                 'co1', 'co2', 'co3', 'c11', 'c12', 'c13'):
        W, b = wb(name)
        wlist += [W, b]
    loss, pred, logits = _final(a0, a1, h5, h6, dn, dinv,
                                x1, x2, z1, z2, gf, y2, wlist)
    return loss.reshape(()), pred[:N, 0], logits[:N]


# speedup vs baseline: 12.0377x; 2.0093x over previous
import functools

import jax
import jax.numpy as jnp
from jax import lax
from jax.experimental import pallas as pl
from jax.experimental.pallas import tpu as pltpu
from jax.experimental.pallas import tpu_sc as plsc

N = 10000
E = 320000
NFEAT = 135
NF = 128
NHID = 128
NCLS = 7

NP = 10240
NC, NS, L = 2, 16, 16
CH = 128
RPS = 160
EROWS = NS * RPS
EPAD = EROWS * CH
EPW = EPAD // (NC * NS)
ROWS_N = NP // NS
NBUF = 4

BR = 512
NPROG = NP // BR


_DROWS = EROWS // (NC * NS)


def _deg_body(dst3, ones_hbm, zeros_hbm, out, dstv, ones_v, acc):
    c = lax.axis_index("c")
    s = lax.axis_index("s")
    row0 = (c * NS + s) * _DROWS
    nb = s * ROWS_N
    pltpu.sync_copy(dst3.at[pl.ds(row0, _DROWS)], dstv)
    pltpu.sync_copy(ones_hbm, ones_v)
    pltpu.sync_copy(zeros_hbm.at[pl.ds(nb, ROWS_N)], acc.at[pl.ds(nb, ROWS_N)])
    plsc.subcore_barrier()

    def body(j, _):
        pltpu.sync_copy(ones_v, acc.at[dstv.at[j]], add=True)
        return 0

    lax.fori_loop(0, _DROWS, body, 0)
    plsc.subcore_barrier()
    pltpu.sync_copy(acc.at[pl.ds(nb, ROWS_N)], out.at[pl.ds(c * NP + nb, ROWS_N)])


GRP = 8
NG = RPS // GRP


def _agg_body(edges3, h0, h1, zeros_hbm, out0, out1, ibuf, gbuf, acc,
              isem, gsem, ssem):
    c = lax.axis_index("c")
    s = lax.axis_index("s")
    row0 = s * RPS
    nb = s * ROWS_N
    pltpu.sync_copy(zeros_hbm.at[pl.ds(nb, ROWS_N)], acc.at[pl.ds(nb, ROWS_N)])
    plsc.subcore_barrier()

    def run(h, out):
        def icopy(g):
            pltpu.async_copy(edges3.at[pl.ds(row0 + (g % NG) * GRP, GRP)],
                             ibuf.at[g % 2], isem)

        def iwait(g):
            pltpu.make_async_copy(edges3.at[pl.ds(row0, GRP)],
                                  ibuf.at[g % 2], isem).wait()

        def gstart(gm, k):
            pltpu.async_copy(h.at[ibuf.at[gm, k, 0]],
                             gbuf.at[k % 2], gsem.at[k % 2])

        def gwait(gm, k):
            pltpu.make_async_copy(h.at[ibuf.at[gm, k, 0]],
                                  gbuf.at[k % 2], gsem.at[k % 2]).wait()

        def sstart(gm, k):
            pltpu.async_copy(gbuf.at[k % 2], acc.at[ibuf.at[gm, k, 1]],
                             ssem.at[k % 2], add=True)

        def swait(gm, k):
            pltpu.make_async_copy(gbuf.at[k % 2], acc.at[ibuf.at[gm, k, 1]],
                                  ssem.at[k % 2]).wait()

        icopy(0)

        def body(g, _):
            gm = g % 2
            iwait(g)
            icopy(g + 1)
            gstart(gm, 0)
            gstart(gm, 1)
            for k in range(GRP):
                gwait(gm, k)
                sstart(gm, k)
                if k + 2 < GRP:
                    swait(gm, k)
                    gstart(gm, k + 2)
            swait(gm, GRP - 2)
            swait(gm, GRP - 1)
            return 0

        lax.fori_loop(0, NG, body, 0)
        iwait(NG)
        plsc.subcore_barrier()
        pltpu.sync_copy(acc.at[pl.ds(nb, ROWS_N)], out.at[pl.ds(nb, ROWS_N)])

    @pl.when(c == 0)
    def _():
        run(h0, out0)

    @pl.when(c == 1)
    def _():
        run(h1, out1)


@functools.cache
def _get_sc_kernels():
    mesh = plsc.VectorSubcoreMesh(core_axis_name="c", subcore_axis_name="s",
                                  num_cores=NC, num_subcores=NS)
    deg = pl.kernel(
        _deg_body,
        out_type=jax.ShapeDtypeStruct((NC * NP, NHID), jnp.float32),
        mesh=mesh,
        scratch_types=[
            pltpu.VMEM((_DROWS, CH), jnp.int32),
            pltpu.VMEM((CH, NHID), jnp.float32),
            pltpu.VMEM_SHARED((NP, NHID), jnp.float32),
        ],
    )
    agg = pl.kernel(
        _agg_body,
        out_type=[jax.ShapeDtypeStruct((NP, NHID), jnp.float32)] * 2,
        mesh=mesh,
        scratch_types=[
            pltpu.VMEM((2, GRP, 2, CH), jnp.int32),
            pltpu.VMEM((2, CH, NHID), jnp.float32),
            pltpu.VMEM_SHARED((NP, NHID), jnp.float32),
            pltpu.SemaphoreType.DMA,
            pltpu.SemaphoreType.DMA((2,)),
            pltpu.SemaphoreType.DMA((2,)),
        ],
    )
    return deg, agg


def _dot(a, b):
    return jnp.dot(a, b, preferred_element_type=jnp.float32)


def _lin1_body(x_ref, w1_ref, b1_ref, w4_ref, b4_ref,
               h1_ref, h2_ref, dinv_ref):
    x = x_ref[...]
    h1_ref[...] = _dot(x, w1_ref[...]) + b1_ref[...]
    h2_ref[...] = _dot(x, w4_ref[...]) + b4_ref[...]
    dinv_ref[...] = 1.0 / (x[:, NF:NF + 1] + 1.0)


def _scale_body(p0_ref, p1_ref, h1_ref, h2_ref, dn_ref, h1s_ref, h2s_ref):
    d = p0_ref[:, 0:1] + p1_ref[:, 0:1] + 1.0
    dn = lax.rsqrt(d)
    dn_ref[...] = dn
    h1s_ref[...] = h1_ref[...] * dn
    h2s_ref[...] = h2_ref[...] * dn


def _mid_body(a0_ref, a1_ref, hp1_ref, hp2_ref, dn_ref, dinv_ref,
              wa_ref, ba_ref, wb_ref, bb_ref,
              u1_ref, u2_ref, g1_ref, g2_ref, g1s_ref, g2s_ref):
    dn = dn_ref[...]
    dinv = dinv_ref[...]
    u1 = jnp.maximum(dn * a0_ref[...] + hp2_ref[...] * dinv, 0.0)
    u2 = jnp.maximum(dn * a1_ref[...] + hp1_ref[...] * dinv, 0.0)
    g1 = _dot(u1, wa_ref[...]) + ba_ref[...]
    g2 = _dot(u2, wb_ref[...]) + bb_ref[...]
    u1_ref[...] = u1
    u2_ref[...] = u2
    g1_ref[...] = g1
    g2_ref[...] = g2
    g1s_ref[...] = g1 * dn
    g2s_ref[...] = g2 * dn


def _log_softmax(l):
    m = jnp.max(l, axis=1, keepdims=True)
    e = jnp.exp(l - m)
    return l - m - jnp.log(jnp.sum(e, axis=1, keepdims=True))


def _final_body(a0_ref, a1_ref, h5_ref, h6_ref, dn_ref, dinv_ref,
                x1_ref, x2_ref, z1_ref, z2_ref, gf_ref, y_ref,
                fl1w_ref, fl1b_ref, fl2w_ref, fl2b_ref, fl3w_ref, fl3b_ref,
                fl4w_ref, fl4b_ref, fl5w_ref, fl5b_ref, fl6w_ref, fl6b_ref,
                co1w_ref, co1b_ref, co2w_ref, co2b_ref, co3w_ref, co3b_ref,
                c11w_ref, c11b_ref, c12w_ref, c12b_ref, c13w_ref, c13b_ref,
                loss_ref, pred_ref, logits_ref, acc_ref):
    pid = pl.program_id(0)
    dn = dn_ref[...]
    dinv = dinv_ref[...]
    v1 = jnp.maximum(dn * a0_ref[...] + h6_ref[...] * dinv, 0.0)
    v2 = jnp.maximum(dn * a1_ref[...] + h5_ref[...] * dinv, 0.0)
    x1, x2, z1, z2 = x1_ref[...], x2_ref[...], z1_ref[...], z2_ref[...]

    def cat6(w_ref, parts, sizes):
        o = 0
        acc = None
        for p, sz in zip(parts, sizes):
            term = _dot(p, w_ref[pl.ds(o, sz), :])
            acc = term if acc is None else acc + term
            o += sz
        return acc

    nf_parts = (x1, z1, v1, x2, z2, v2)
    nf_sizes = (128, 128, 128, 128, 128, 128)
    a1 = jnp.maximum(cat6(fl1w_ref, nf_parts, nf_sizes) + fl1b_ref[...], 0.0)
    a2 = jnp.maximum(_dot(a1, fl2w_ref[...]) + fl2b_ref[...], 0.0)
    a3 = jnp.maximum(_dot(a2, fl3w_ref[...]) + fl3b_ref[...], 0.0)
    gf = gf_ref[...]
    b1 = jnp.maximum(_dot(gf, fl4w_ref[...]) + fl4b_ref[...], 0.0)
    b2 = jnp.maximum(_dot(b1, fl5w_ref[...]) + fl5b_ref[...], 0.0)
    b3 = jnp.maximum(_dot(b2, fl6w_ref[...]) + fl6b_ref[...], 0.0)

    fea_parts = (a1, a2, a3, b1, b2, b3)
    fea_sizes = (256, 128, 128, 64, 32, 32)
    hO = jnp.maximum(cat6(co1w_ref, fea_parts, fea_sizes) + co1b_ref[...], 0.0)
    hO2 = jnp.maximum(_dot(hO, co2w_ref[...]) + co2b_ref[...], 0.0)
    logitsO = _dot(hO2, co3w_ref[...]) + co3b_ref[...]

    h1c = jnp.maximum(cat6(c11w_ref, nf_parts, nf_sizes) + c11b_ref[...], 0.0)
    h2c = jnp.maximum(_dot(h1c, c12w_ref[...]) + c12b_ref[...], 0.0)
    logits1 = _dot(h2c, c13w_ref[...]) + c13b_ref[...]

    lpO = _log_softmax(logitsO)
    lp1 = _log_softmax(logits1)
    cls = lax.broadcasted_iota(jnp.int32, (BR, NCLS), 1)
    onehot = (cls == y_ref[...]).astype(jnp.float32)
    rid = pid * BR + lax.broadcasted_iota(jnp.int32, (BR, 1), 0)
    mask = (rid < N).astype(jnp.float32)
    partO = jnp.sum(jnp.sum(lpO * onehot, axis=1, keepdims=True) * mask)
    part1 = jnp.sum(jnp.sum(lp1 * onehot, axis=1, keepdims=True) * mask)

    @pl.when(pid == 0)
    def _():
        acc_ref[0] = 0.0
        acc_ref[1] = 0.0

    acc_ref[0] += partO
    acc_ref[1] += part1

    @pl.when(pid == NPROG - 1)
    def _():
        loss_ref[...] = jnp.full((1, 1), -(acc_ref[0] + acc_ref[1]) / N,
                                 jnp.float32)

    m = logitsO[:, 0:1]
    idx = jnp.zeros((BR, 1), jnp.int32)
    for cc in range(1, NCLS):
        col = logitsO[:, cc:cc + 1]
        gt = col > m
        idx = jnp.where(gt, cc, idx)
        m = jnp.where(gt, col, m)
    pred_ref[...] = idx
    logits_ref[...] = logitsO


def _row_spec(cols):
    return pl.BlockSpec((BR, cols), lambda i: (i, 0))


def _full_spec(shape):
    nd = len(shape)
    return pl.BlockSpec(shape, lambda i, _n=nd: (0,) * nd)


def _lin1(xp, w1, b1, w4, b4):
    return pl.pallas_call(
        _lin1_body,
        grid=(NPROG,),
        in_specs=[
            _row_spec(NFEAT),
            _full_spec(w1.shape), _full_spec(b1.shape),
            _full_spec(w4.shape), _full_spec(b4.shape),
        ],
        out_specs=[_row_spec(NHID), _row_spec(NHID), _row_spec(1)],
        out_shape=[
            jax.ShapeDtypeStruct((NP, NHID), jnp.float32),
            jax.ShapeDtypeStruct((NP, NHID), jnp.float32),
            jax.ShapeDtypeStruct((NP, 1), jnp.float32),
        ],
    )(xp, w1, b1, w4, b4)


_BRS = 2048


def _scale(p0, p1, h1, h2):
    return pl.pallas_call(
        _scale_body,
        grid=(NP // _BRS,),
        in_specs=[
            pl.BlockSpec((_BRS, NHID), lambda i: (i, 0)),
            pl.BlockSpec((_BRS, NHID), lambda i: (i, 0)),
            pl.BlockSpec((_BRS, NHID), lambda i: (i, 0)),
            pl.BlockSpec((_BRS, NHID), lambda i: (i, 0)),
        ],
        out_specs=[
            pl.BlockSpec((_BRS, 1), lambda i: (i, 0)),
            pl.BlockSpec((_BRS, NHID), lambda i: (i, 0)),
            pl.BlockSpec((_BRS, NHID), lambda i: (i, 0)),
        ],
        out_shape=[
            jax.ShapeDtypeStruct((NP, 1), jnp.float32),
            jax.ShapeDtypeStruct((NP, NHID), jnp.float32),
            jax.ShapeDtypeStruct((NP, NHID), jnp.float32),
        ],
    )(p0, p1, h1, h2)


def _mid(a0, a1, hp1, hp2, dn, dinv, wa, ba, wb, bb):
    return pl.pallas_call(
        _mid_body,
        grid=(NPROG,),
        in_specs=[
            _row_spec(NHID), _row_spec(NHID),
            _row_spec(NHID), _row_spec(NHID),
            _row_spec(1), _row_spec(1),
            _full_spec(wa.shape), _full_spec(ba.shape),
            _full_spec(wb.shape), _full_spec(bb.shape),
        ],
        out_specs=[_row_spec(NHID)] * 6,
        out_shape=[jax.ShapeDtypeStruct((NP, NHID), jnp.float32)] * 6,
    )(a0, a1, hp1, hp2, dn, dinv, wa, ba, wb, bb)


def _final(a0, a1, h5, h6, dn, dinv, x1, x2, z1, z2, gf, y2, wlist):
    in_specs = [
        _row_spec(NHID), _row_spec(NHID),
        _row_spec(NHID), _row_spec(NHID),
        _row_spec(1), _row_spec(1),
        _row_spec(NHID), _row_spec(NHID),
        _row_spec(NHID), _row_spec(NHID),
        _row_spec(6), _row_spec(1),
    ] + [_full_spec(w.shape) for w in wlist]
    return pl.pallas_call(
        _final_body,
        grid=(NPROG,),
        in_specs=in_specs,
        out_specs=[
            pl.BlockSpec((1, 1), lambda i: (0, 0)),
            _row_spec(1),
            _row_spec(NCLS),
        ],
        out_shape=[
            jax.ShapeDtypeStruct((1, 1), jnp.float32),
            jax.ShapeDtypeStruct((NP, 1), jnp.int32),
            jax.ShapeDtypeStruct((NP, NCLS), jnp.float32),
        ],
        scratch_shapes=[pltpu.SMEM((2,), jnp.float32)],
    )(a0, a1, h5, h6, dn, dinv, x1, x2, z1, z2, gf, y2, *wlist)


def kernel(x_, edge_index, y, params):
    f32 = jnp.float32
    xp = jnp.pad(x_, ((0, NP - N), (0, 0)))
    gf = xp[:, NF + 1:]
    y2 = jnp.pad(y, (0, NP - N)).reshape(NP, 1)

    src = edge_index[0]
    dst = edge_index[1]
    epad = N + (jnp.arange(EPAD - E, dtype=jnp.int32) % (NP - N))
    src3 = jnp.concatenate([src, epad]).reshape(EROWS, CH)
    dst3 = jnp.concatenate([dst, epad]).reshape(EROWS, CH)
    edges3 = jnp.stack([src3, dst3], axis=1)
    zeros128 = jnp.zeros((NP, NHID), f32)

    def wb(name):
        W, b = params[name]
        return W, b.reshape(1, -1)

    w1, b1 = wb('conv1')
    w4, b4 = wb('conv4')

    _deg_kernel, _agg_kernel = _get_sc_kernels()
    ones128 = jnp.ones((CH, NHID), f32)
    deg_out = _deg_kernel(dst3, ones128, zeros128)
    p0, p1 = deg_out[:NP], deg_out[NP:]
    h1, h2, dinv = _lin1(xp, w1, b1, w4, b4)
    dn, h1s, h2s = _scale(p0, p1, h1, h2)

    w2, b2 = wb('conv2')
    w5, b5 = wb('conv5')
    a0, a1 = _agg_kernel(edges3, h1s, h2s, zeros128)
    x1, x2, h3, h4, h3s, h4s = _mid(a0, a1, h1, h2, dn, dinv, w2, b2, w5, b5)

    w3, b3 = wb('conv3')
    w6, b6 = wb('conv6')
    a0, a1 = _agg_kernel(edges3, h3s, h4s, zeros128)
    z1, z2, h5, h6, h5s, h6s = _mid(a0, a1, h3, h4, dn, dinv, w3, b3, w6, b6)

    a0, a1 = _agg_kernel(edges3, h5s, h6s, zeros128)

    wlist = []
    for name in ('fl1', 'fl2', 'fl3', 'fl4', 'fl5', 'fl6',
                 'co1', 'co2', 'co3', 'c11', 'c12', 'c13'):
        W, b = wb(name)
        wlist += [W, b]
    loss, pred, logits = _final(a0, a1, h5, h6, dn, dinv,
                                x1, x2, z1, z2, gf, y2, wlist)
    return loss.reshape(()), pred[:N, 0], logits[:N]


# speedup vs baseline: 12.5019x; 1.0386x over previous
import functools

import jax
import jax.numpy as jnp
from jax import lax
from jax.experimental import pallas as pl
from jax.experimental.pallas import tpu as pltpu
from jax.experimental.pallas import tpu_sc as plsc

N = 10000
E = 320000
NFEAT = 135
NF = 128
NHID = 128
NCLS = 7

NP = 10240
NC, NS, L = 2, 16, 16
CH = 128
RPS = 160
EROWS = NS * RPS
EPAD = EROWS * CH
EPW = EPAD // (NC * NS)
ROWS_N = NP // NS
NBUF = 4

BR = 512
NPROG = NP // BR


_DROWS = EROWS // (NC * NS)


def _deg_body(dst3, ones_hbm, zeros_hbm, out, dstv, ones_v, acc):
    c = lax.axis_index("c")
    s = lax.axis_index("s")
    row0 = (c * NS + s) * _DROWS
    nb = s * ROWS_N
    pltpu.sync_copy(dst3.at[pl.ds(row0, _DROWS)], dstv)
    pltpu.sync_copy(ones_hbm, ones_v)
    pltpu.sync_copy(zeros_hbm.at[pl.ds(nb, ROWS_N)], acc.at[pl.ds(nb, ROWS_N)])
    plsc.subcore_barrier()

    def body(j, _):
        pltpu.sync_copy(ones_v, acc.at[dstv.at[j]], add=True)
        return 0

    lax.fori_loop(0, _DROWS, body, 0)
    plsc.subcore_barrier()
    pltpu.sync_copy(acc.at[pl.ds(nb, ROWS_N)], out.at[pl.ds(c * NP + nb, ROWS_N)])


GRP = 16
NG = RPS // GRP


def _agg_body(edges3, h0, h1, zeros_hbm, out0, out1, ibuf, gbuf, acc,
              isem, gsem, ssem):
    c = lax.axis_index("c")
    s = lax.axis_index("s")
    row0 = s * RPS
    nb = s * ROWS_N
    pltpu.sync_copy(zeros_hbm.at[pl.ds(nb, ROWS_N)], acc.at[pl.ds(nb, ROWS_N)])
    plsc.subcore_barrier()

    def run(h, out):
        def icopy(g):
            pltpu.async_copy(edges3.at[pl.ds(row0 + (g % NG) * GRP, GRP)],
                             ibuf.at[g % 2], isem)

        def iwait(g):
            pltpu.make_async_copy(edges3.at[pl.ds(row0, GRP)],
                                  ibuf.at[g % 2], isem).wait()

        def gstart(gm, k):
            pltpu.async_copy(h.at[ibuf.at[gm, k, 0]],
                             gbuf.at[k % 2], gsem.at[k % 2])

        def gwait(gm, k):
            pltpu.make_async_copy(h.at[ibuf.at[gm, k, 0]],
                                  gbuf.at[k % 2], gsem.at[k % 2]).wait()

        def sstart(gm, k):
            pltpu.async_copy(gbuf.at[k % 2], acc.at[ibuf.at[gm, k, 1]],
                             ssem.at[k % 2], add=True)

        def swait(gm, k):
            pltpu.make_async_copy(gbuf.at[k % 2], acc.at[ibuf.at[gm, k, 1]],
                                  ssem.at[k % 2]).wait()

        icopy(0)

        def body(g, _):
            gm = g % 2
            iwait(g)
            icopy(g + 1)
            gstart(gm, 0)
            gstart(gm, 1)
            for k in range(GRP):
                gwait(gm, k)
                sstart(gm, k)
                if k + 2 < GRP:
                    swait(gm, k)
                    gstart(gm, k + 2)
            swait(gm, GRP - 2)
            swait(gm, GRP - 1)
            return 0

        lax.fori_loop(0, NG, body, 0)
        iwait(NG)
        plsc.subcore_barrier()
        pltpu.sync_copy(acc.at[pl.ds(nb, ROWS_N)], out.at[pl.ds(nb, ROWS_N)])

    @pl.when(c == 0)
    def _():
        run(h0, out0)

    @pl.when(c == 1)
    def _():
        run(h1, out1)


@functools.cache
def _get_sc_kernels():
    mesh = plsc.VectorSubcoreMesh(core_axis_name="c", subcore_axis_name="s",
                                  num_cores=NC, num_subcores=NS)
    deg = pl.kernel(
        _deg_body,
        out_type=jax.ShapeDtypeStruct((NC * NP, NHID), jnp.float32),
        mesh=mesh,
        scratch_types=[
            pltpu.VMEM((_DROWS, CH), jnp.int32),
            pltpu.VMEM((CH, NHID), jnp.float32),
            pltpu.VMEM_SHARED((NP, NHID), jnp.float32),
        ],
    )
    agg = pl.kernel(
        _agg_body,
        out_type=[jax.ShapeDtypeStruct((NP, NHID), jnp.float32)] * 2,
        mesh=mesh,
        scratch_types=[
            pltpu.VMEM((2, GRP, 2, CH), jnp.int32),
            pltpu.VMEM((2, CH, NHID), jnp.float32),
            pltpu.VMEM_SHARED((NP, NHID), jnp.float32),
            pltpu.SemaphoreType.DMA,
            pltpu.SemaphoreType.DMA((2,)),
            pltpu.SemaphoreType.DMA((2,)),
        ],
    )
    return deg, agg


def _dot(a, b):
    return jnp.dot(a, b, preferred_element_type=jnp.float32)


def _lin1_body(x_ref, w1_ref, b1_ref, w4_ref, b4_ref,
               h1_ref, h2_ref, dinv_ref):
    x = x_ref[...]
    h1_ref[...] = _dot(x, w1_ref[...]) + b1_ref[...]
    h2_ref[...] = _dot(x, w4_ref[...]) + b4_ref[...]
    dinv_ref[...] = 1.0 / (x[:, NF:NF + 1] + 1.0)


def _scale_body(p0_ref, p1_ref, h1_ref, h2_ref, dn_ref, h1s_ref, h2s_ref):
    d = p0_ref[:, 0:1] + p1_ref[:, 0:1] + 1.0
    dn = lax.rsqrt(d)
    dn_ref[...] = dn
    h1s_ref[...] = h1_ref[...] * dn
    h2s_ref[...] = h2_ref[...] * dn


def _mid_body(a0_ref, a1_ref, hp1_ref, hp2_ref, dn_ref, dinv_ref,
              wa_ref, ba_ref, wb_ref, bb_ref,
              u1_ref, u2_ref, g1_ref, g2_ref, g1s_ref, g2s_ref):
    dn = dn_ref[...]
    dinv = dinv_ref[...]
    u1 = jnp.maximum(dn * a0_ref[...] + hp2_ref[...] * dinv, 0.0)
    u2 = jnp.maximum(dn * a1_ref[...] + hp1_ref[...] * dinv, 0.0)
    g1 = _dot(u1, wa_ref[...]) + ba_ref[...]
    g2 = _dot(u2, wb_ref[...]) + bb_ref[...]
    u1_ref[...] = u1
    u2_ref[...] = u2
    g1_ref[...] = g1
    g2_ref[...] = g2
    g1s_ref[...] = g1 * dn
    g2s_ref[...] = g2 * dn


def _log_softmax(l):
    m = jnp.max(l, axis=1, keepdims=True)
    e = jnp.exp(l - m)
    return l - m - jnp.log(jnp.sum(e, axis=1, keepdims=True))


def _final_body(a0_ref, a1_ref, h5_ref, h6_ref, dn_ref, dinv_ref,
                x1_ref, x2_ref, z1_ref, z2_ref, gf_ref, y_ref,
                fl1w_ref, fl1b_ref, fl2w_ref, fl2b_ref, fl3w_ref, fl3b_ref,
                fl4w_ref, fl4b_ref, fl5w_ref, fl5b_ref, fl6w_ref, fl6b_ref,
                co1w_ref, co1b_ref, co2w_ref, co2b_ref, co3w_ref, co3b_ref,
                c11w_ref, c11b_ref, c12w_ref, c12b_ref, c13w_ref, c13b_ref,
                loss_ref, pred_ref, logits_ref, acc_ref):
    pid = pl.program_id(0)
    dn = dn_ref[...]
    dinv = dinv_ref[...]
    v1 = jnp.maximum(dn * a0_ref[...] + h6_ref[...] * dinv, 0.0)
    v2 = jnp.maximum(dn * a1_ref[...] + h5_ref[...] * dinv, 0.0)
    x1, x2, z1, z2 = x1_ref[...], x2_ref[...], z1_ref[...], z2_ref[...]

    def cat6(w_ref, parts, sizes):
        o = 0
        acc = None
        for p, sz in zip(parts, sizes):
            term = _dot(p, w_ref[pl.ds(o, sz), :])
            acc = term if acc is None else acc + term
            o += sz
        return acc

    nf_parts = (x1, z1, v1, x2, z2, v2)
    nf_sizes = (128, 128, 128, 128, 128, 128)
    a1 = jnp.maximum(cat6(fl1w_ref, nf_parts, nf_sizes) + fl1b_ref[...], 0.0)
    a2 = jnp.maximum(_dot(a1, fl2w_ref[...]) + fl2b_ref[...], 0.0)
    a3 = jnp.maximum(_dot(a2, fl3w_ref[...]) + fl3b_ref[...], 0.0)
    gf = gf_ref[...]
    b1 = jnp.maximum(_dot(gf, fl4w_ref[...]) + fl4b_ref[...], 0.0)
    b2 = jnp.maximum(_dot(b1, fl5w_ref[...]) + fl5b_ref[...], 0.0)
    b3 = jnp.maximum(_dot(b2, fl6w_ref[...]) + fl6b_ref[...], 0.0)

    fea_parts = (a1, a2, a3, b1, b2, b3)
    fea_sizes = (256, 128, 128, 64, 32, 32)
    hO = jnp.maximum(cat6(co1w_ref, fea_parts, fea_sizes) + co1b_ref[...], 0.0)
    hO2 = jnp.maximum(_dot(hO, co2w_ref[...]) + co2b_ref[...], 0.0)
    logitsO = _dot(hO2, co3w_ref[...]) + co3b_ref[...]

    h1c = jnp.maximum(cat6(c11w_ref, nf_parts, nf_sizes) + c11b_ref[...], 0.0)
    h2c = jnp.maximum(_dot(h1c, c12w_ref[...]) + c12b_ref[...], 0.0)
    logits1 = _dot(h2c, c13w_ref[...]) + c13b_ref[...]

    lpO = _log_softmax(logitsO)
    lp1 = _log_softmax(logits1)
    cls = lax.broadcasted_iota(jnp.int32, (BR, NCLS), 1)
    onehot = (cls == y_ref[...]).astype(jnp.float32)
    rid = pid * BR + lax.broadcasted_iota(jnp.int32, (BR, 1), 0)
    mask = (rid < N).astype(jnp.float32)
    partO = jnp.sum(jnp.sum(lpO * onehot, axis=1, keepdims=True) * mask)
    part1 = jnp.sum(jnp.sum(lp1 * onehot, axis=1, keepdims=True) * mask)

    @pl.when(pid == 0)
    def _():
        acc_ref[0] = 0.0
        acc_ref[1] = 0.0

    acc_ref[0] += partO
    acc_ref[1] += part1

    @pl.when(pid == NPROG - 1)
    def _():
        loss_ref[...] = jnp.full((1, 1), -(acc_ref[0] + acc_ref[1]) / N,
                                 jnp.float32)

    m = logitsO[:, 0:1]
    idx = jnp.zeros((BR, 1), jnp.int32)
    for cc in range(1, NCLS):
        col = logitsO[:, cc:cc + 1]
        gt = col > m
        idx = jnp.where(gt, cc, idx)
        m = jnp.where(gt, col, m)
    pred_ref[...] = idx
    logits_ref[...] = logitsO


def _row_spec(cols):
    return pl.BlockSpec((BR, cols), lambda i: (i, 0))


def _full_spec(shape):
    nd = len(shape)
    return pl.BlockSpec(shape, lambda i, _n=nd: (0,) * nd)


def _lin1(xp, w1, b1, w4, b4):
    return pl.pallas_call(
        _lin1_body,
        grid=(NPROG,),
        in_specs=[
            _row_spec(NFEAT),
            _full_spec(w1.shape), _full_spec(b1.shape),
            _full_spec(w4.shape), _full_spec(b4.shape),
        ],
        out_specs=[_row_spec(NHID), _row_spec(NHID), _row_spec(1)],
        out_shape=[
            jax.ShapeDtypeStruct((NP, NHID), jnp.float32),
            jax.ShapeDtypeStruct((NP, NHID), jnp.float32),
            jax.ShapeDtypeStruct((NP, 1), jnp.float32),
        ],
    )(xp, w1, b1, w4, b4)


_BRS = 2048


def _scale(p0, p1, h1, h2):
    return pl.pallas_call(
        _scale_body,
        grid=(NP // _BRS,),
        in_specs=[
            pl.BlockSpec((_BRS, NHID), lambda i: (i, 0)),
            pl.BlockSpec((_BRS, NHID), lambda i: (i, 0)),
            pl.BlockSpec((_BRS, NHID), lambda i: (i, 0)),
            pl.BlockSpec((_BRS, NHID), lambda i: (i, 0)),
        ],
        out_specs=[
            pl.BlockSpec((_BRS, 1), lambda i: (i, 0)),
            pl.BlockSpec((_BRS, NHID), lambda i: (i, 0)),
            pl.BlockSpec((_BRS, NHID), lambda i: (i, 0)),
        ],
        out_shape=[
            jax.ShapeDtypeStruct((NP, 1), jnp.float32),
            jax.ShapeDtypeStruct((NP, NHID), jnp.float32),
            jax.ShapeDtypeStruct((NP, NHID), jnp.float32),
        ],
    )(p0, p1, h1, h2)


def _mid(a0, a1, hp1, hp2, dn, dinv, wa, ba, wb, bb):
    return pl.pallas_call(
        _mid_body,
        grid=(NPROG,),
        in_specs=[
            _row_spec(NHID), _row_spec(NHID),
            _row_spec(NHID), _row_spec(NHID),
            _row_spec(1), _row_spec(1),
            _full_spec(wa.shape), _full_spec(ba.shape),
            _full_spec(wb.shape), _full_spec(bb.shape),
        ],
        out_specs=[_row_spec(NHID)] * 6,
        out_shape=[jax.ShapeDtypeStruct((NP, NHID), jnp.float32)] * 6,
    )(a0, a1, hp1, hp2, dn, dinv, wa, ba, wb, bb)


def _final(a0, a1, h5, h6, dn, dinv, x1, x2, z1, z2, gf, y2, wlist):
    in_specs = [
        _row_spec(NHID), _row_spec(NHID),
        _row_spec(NHID), _row_spec(NHID),
        _row_spec(1), _row_spec(1),
        _row_spec(NHID), _row_spec(NHID),
        _row_spec(NHID), _row_spec(NHID),
        _row_spec(6), _row_spec(1),
    ] + [_full_spec(w.shape) for w in wlist]
    return pl.pallas_call(
        _final_body,
        grid=(NPROG,),
        in_specs=in_specs,
        out_specs=[
            pl.BlockSpec((1, 1), lambda i: (0, 0)),
            _row_spec(1),
            _row_spec(NCLS),
        ],
        out_shape=[
            jax.ShapeDtypeStruct((1, 1), jnp.float32),
            jax.ShapeDtypeStruct((NP, 1), jnp.int32),
            jax.ShapeDtypeStruct((NP, NCLS), jnp.float32),
        ],
        scratch_shapes=[pltpu.SMEM((2,), jnp.float32)],
    )(a0, a1, h5, h6, dn, dinv, x1, x2, z1, z2, gf, y2, *wlist)


def kernel(x_, edge_index, y, params):
    f32 = jnp.float32
    xp = jnp.pad(x_, ((0, NP - N), (0, 0)))
    gf = xp[:, NF + 1:]
    y2 = jnp.pad(y, (0, NP - N)).reshape(NP, 1)

    src = edge_index[0]
    dst = edge_index[1]
    epad = N + (jnp.arange(EPAD - E, dtype=jnp.int32) % (NP - N))
    src3 = jnp.concatenate([src, epad]).reshape(EROWS, CH)
    dst3 = jnp.concatenate([dst, epad]).reshape(EROWS, CH)
    edges3 = jnp.stack([src3, dst3], axis=1)
    zeros128 = jnp.zeros((NP, NHID), f32)

    def wb(name):
        W, b = params[name]
        return W, b.reshape(1, -1)

    w1, b1 = wb('conv1')
    w4, b4 = wb('conv4')

    _deg_kernel, _agg_kernel = _get_sc_kernels()
    ones128 = jnp.ones((CH, NHID), f32)
    deg_out = _deg_kernel(dst3, ones128, zeros128)
    p0, p1 = deg_out[:NP], deg_out[NP:]
    h1, h2, dinv = _lin1(xp, w1, b1, w4, b4)
    dn, h1s, h2s = _scale(p0, p1, h1, h2)

    w2, b2 = wb('conv2')
    w5, b5 = wb('conv5')
    a0, a1 = _agg_kernel(edges3, h1s, h2s, zeros128)
    x1, x2, h3, h4, h3s, h4s = _mid(a0, a1, h1, h2, dn, dinv, w2, b2, w5, b5)

    w3, b3 = wb('conv3')
    w6, b6 = wb('conv6')
    a0, a1 = _agg_kernel(edges3, h3s, h4s, zeros128)
    z1, z2, h5, h6, h5s, h6s = _mid(a0, a1, h3, h4, dn, dinv, w3, b3, w6, b6)

    a0, a1 = _agg_kernel(edges3, h5s, h6s, zeros128)

    wlist = []
    for name in ('fl1', 'fl2', 'fl3', 'fl4', 'fl5', 'fl6',
                 'co1', 'co2', 'co3', 'c11', 'c12', 'c13'):
        W, b = wb(name)
        wlist += [W, b]
    loss, pred, logits = _final(a0, a1, h5, h6, dn, dinv,
                                x1, x2, z1, z2, gf, y2, wlist)
    return loss.reshape(()), pred[:N, 0], logits[:N]


# speedup vs baseline: 12.7120x; 1.0168x over previous
import functools

import jax
import jax.numpy as jnp
from jax import lax
from jax.experimental import pallas as pl
from jax.experimental.pallas import tpu as pltpu
from jax.experimental.pallas import tpu_sc as plsc

N = 10000
E = 320000
NFEAT = 135
NF = 128
NHID = 128
NCLS = 7

NP = 10240
NC, NS, L = 2, 16, 16
CH = 128
RPS = 160
EROWS = NS * RPS
EPAD = EROWS * CH
EPW = EPAD // (NC * NS)
ROWS_N = NP // NS
NBUF = 4

BR = 512
NPROG = NP // BR


_DROWS = EROWS // (NC * NS)


def _deg_body(dst3, ones_hbm, zeros_hbm, out, dstv, ones_v, acc):
    c = lax.axis_index("c")
    s = lax.axis_index("s")
    row0 = (c * NS + s) * _DROWS
    nb = s * ROWS_N
    pltpu.sync_copy(dst3.at[pl.ds(row0, _DROWS)], dstv)
    pltpu.sync_copy(ones_hbm, ones_v)
    pltpu.sync_copy(zeros_hbm.at[pl.ds(nb, ROWS_N)], acc.at[pl.ds(nb, ROWS_N)])
    plsc.subcore_barrier()

    def body(j, _):
        pltpu.sync_copy(ones_v, acc.at[dstv.at[j]], add=True)
        return 0

    lax.fori_loop(0, _DROWS, body, 0)
    plsc.subcore_barrier()
    pltpu.sync_copy(acc.at[pl.ds(nb, ROWS_N)], out.at[pl.ds(c * NP + nb, ROWS_N)])


GRP = 32
NG = RPS // GRP


def _agg_body(edges3, h0, h1, zeros_hbm, out0, out1, ibuf, gbuf, acc,
              isem, gsem, ssem):
    c = lax.axis_index("c")
    s = lax.axis_index("s")
    row0 = s * RPS
    nb = s * ROWS_N
    pltpu.sync_copy(zeros_hbm.at[pl.ds(nb, ROWS_N)], acc.at[pl.ds(nb, ROWS_N)])
    plsc.subcore_barrier()

    def run(h, out):
        def icopy(g):
            pltpu.async_copy(edges3.at[pl.ds(row0 + (g % NG) * GRP, GRP)],
                             ibuf.at[g % 2], isem)

        def iwait(g):
            pltpu.make_async_copy(edges3.at[pl.ds(row0, GRP)],
                                  ibuf.at[g % 2], isem).wait()

        def gstart(gm, k):
            pltpu.async_copy(h.at[ibuf.at[gm, k, 0]],
                             gbuf.at[k % 2], gsem.at[k % 2])

        def gwait(gm, k):
            pltpu.make_async_copy(h.at[ibuf.at[gm, k, 0]],
                                  gbuf.at[k % 2], gsem.at[k % 2]).wait()

        def sstart(gm, k):
            pltpu.async_copy(gbuf.at[k % 2], acc.at[ibuf.at[gm, k, 1]],
                             ssem.at[k % 2], add=True)

        def swait(gm, k):
            pltpu.make_async_copy(gbuf.at[k % 2], acc.at[ibuf.at[gm, k, 1]],
                                  ssem.at[k % 2]).wait()

        icopy(0)

        def body(g, _):
            gm = g % 2
            iwait(g)
            icopy(g + 1)
            gstart(gm, 0)
            gstart(gm, 1)
            for k in range(GRP):
                gwait(gm, k)
                sstart(gm, k)
                if k + 2 < GRP:
                    swait(gm, k)
                    gstart(gm, k + 2)
            swait(gm, GRP - 2)
            swait(gm, GRP - 1)
            return 0

        lax.fori_loop(0, NG, body, 0)
        iwait(NG)
        plsc.subcore_barrier()
        pltpu.sync_copy(acc.at[pl.ds(nb, ROWS_N)], out.at[pl.ds(nb, ROWS_N)])

    @pl.when(c == 0)
    def _():
        run(h0, out0)

    @pl.when(c == 1)
    def _():
        run(h1, out1)


@functools.cache
def _get_sc_kernels():
    mesh = plsc.VectorSubcoreMesh(core_axis_name="c", subcore_axis_name="s",
                                  num_cores=NC, num_subcores=NS)
    deg = pl.kernel(
        _deg_body,
        out_type=jax.ShapeDtypeStruct((NC * NP, NHID), jnp.float32),
        mesh=mesh,
        scratch_types=[
            pltpu.VMEM((_DROWS, CH), jnp.int32),
            pltpu.VMEM((CH, NHID), jnp.float32),
            pltpu.VMEM_SHARED((NP, NHID), jnp.float32),
        ],
    )
    agg = pl.kernel(
        _agg_body,
        out_type=[jax.ShapeDtypeStruct((NP, NHID), jnp.float32)] * 2,
        mesh=mesh,
        scratch_types=[
            pltpu.VMEM((2, GRP, 2, CH), jnp.int32),
            pltpu.VMEM((2, CH, NHID), jnp.float32),
            pltpu.VMEM_SHARED((NP, NHID), jnp.float32),
            pltpu.SemaphoreType.DMA,
            pltpu.SemaphoreType.DMA((2,)),
            pltpu.SemaphoreType.DMA((2,)),
        ],
    )
    return deg, agg


def _dot(a, b):
    return jnp.dot(a, b, preferred_element_type=jnp.float32)


def _lin1_body(x_ref, w1_ref, b1_ref, w4_ref, b4_ref,
               h1_ref, h2_ref, dinv_ref):
    x = x_ref[...]
    h1_ref[...] = _dot(x, w1_ref[...]) + b1_ref[...]
    h2_ref[...] = _dot(x, w4_ref[...]) + b4_ref[...]
    dinv_ref[...] = 1.0 / (x[:, NF:NF + 1] + 1.0)


def _scale_body(p0_ref, p1_ref, h1_ref, h2_ref, dn_ref, h1s_ref, h2s_ref):
    d = p0_ref[:, 0:1] + p1_ref[:, 0:1] + 1.0
    dn = lax.rsqrt(d)
    dn_ref[...] = dn
    h1s_ref[...] = h1_ref[...] * dn
    h2s_ref[...] = h2_ref[...] * dn


def _mid_body(a0_ref, a1_ref, hp1_ref, hp2_ref, dn_ref, dinv_ref,
              wa_ref, ba_ref, wb_ref, bb_ref,
              u1_ref, u2_ref, g1_ref, g2_ref, g1s_ref, g2s_ref):
    dn = dn_ref[...]
    dinv = dinv_ref[...]
    u1 = jnp.maximum(dn * a0_ref[...] + hp2_ref[...] * dinv, 0.0)
    u2 = jnp.maximum(dn * a1_ref[...] + hp1_ref[...] * dinv, 0.0)
    g1 = _dot(u1, wa_ref[...]) + ba_ref[...]
    g2 = _dot(u2, wb_ref[...]) + bb_ref[...]
    u1_ref[...] = u1
    u2_ref[...] = u2
    g1_ref[...] = g1
    g2_ref[...] = g2
    g1s_ref[...] = g1 * dn
    g2s_ref[...] = g2 * dn


def _log_softmax(l):
    m = jnp.max(l, axis=1, keepdims=True)
    e = jnp.exp(l - m)
    return l - m - jnp.log(jnp.sum(e, axis=1, keepdims=True))


def _final_body(a0_ref, a1_ref, h5_ref, h6_ref, dn_ref, dinv_ref,
                x1_ref, x2_ref, z1_ref, z2_ref, gf_ref, y_ref,
                fl1w_ref, fl1b_ref, fl2w_ref, fl2b_ref, fl3w_ref, fl3b_ref,
                fl4w_ref, fl4b_ref, fl5w_ref, fl5b_ref, fl6w_ref, fl6b_ref,
                co1w_ref, co1b_ref, co2w_ref, co2b_ref, co3w_ref, co3b_ref,
                c11w_ref, c11b_ref, c12w_ref, c12b_ref, c13w_ref, c13b_ref,
                loss_ref, pred_ref, logits_ref, acc_ref):
    pid = pl.program_id(0)
    dn = dn_ref[...]
    dinv = dinv_ref[...]
    v1 = jnp.maximum(dn * a0_ref[...] + h6_ref[...] * dinv, 0.0)
    v2 = jnp.maximum(dn * a1_ref[...] + h5_ref[...] * dinv, 0.0)
    x1, x2, z1, z2 = x1_ref[...], x2_ref[...], z1_ref[...], z2_ref[...]

    def cat6(w_ref, parts, sizes):
        o = 0
        acc = None
        for p, sz in zip(parts, sizes):
            term = _dot(p, w_ref[pl.ds(o, sz), :])
            acc = term if acc is None else acc + term
            o += sz
        return acc

    nf_parts = (x1, z1, v1, x2, z2, v2)
    nf_sizes = (128, 128, 128, 128, 128, 128)
    a1 = jnp.maximum(cat6(fl1w_ref, nf_parts, nf_sizes) + fl1b_ref[...], 0.0)
    a2 = jnp.maximum(_dot(a1, fl2w_ref[...]) + fl2b_ref[...], 0.0)
    a3 = jnp.maximum(_dot(a2, fl3w_ref[...]) + fl3b_ref[...], 0.0)
    gf = gf_ref[...]
    b1 = jnp.maximum(_dot(gf, fl4w_ref[...]) + fl4b_ref[...], 0.0)
    b2 = jnp.maximum(_dot(b1, fl5w_ref[...]) + fl5b_ref[...], 0.0)
    b3 = jnp.maximum(_dot(b2, fl6w_ref[...]) + fl6b_ref[...], 0.0)

    fea_parts = (a1, a2, a3, b1, b2, b3)
    fea_sizes = (256, 128, 128, 64, 32, 32)
    hO = jnp.maximum(cat6(co1w_ref, fea_parts, fea_sizes) + co1b_ref[...], 0.0)
    hO2 = jnp.maximum(_dot(hO, co2w_ref[...]) + co2b_ref[...], 0.0)
    logitsO = _dot(hO2, co3w_ref[...]) + co3b_ref[...]

    h1c = jnp.maximum(cat6(c11w_ref, nf_parts, nf_sizes) + c11b_ref[...], 0.0)
    h2c = jnp.maximum(_dot(h1c, c12w_ref[...]) + c12b_ref[...], 0.0)
    logits1 = _dot(h2c, c13w_ref[...]) + c13b_ref[...]

    lpO = _log_softmax(logitsO)
    lp1 = _log_softmax(logits1)
    cls = lax.broadcasted_iota(jnp.int32, (BR, NCLS), 1)
    onehot = (cls == y_ref[...]).astype(jnp.float32)
    rid = pid * BR + lax.broadcasted_iota(jnp.int32, (BR, 1), 0)
    mask = (rid < N).astype(jnp.float32)
    partO = jnp.sum(jnp.sum(lpO * onehot, axis=1, keepdims=True) * mask)
    part1 = jnp.sum(jnp.sum(lp1 * onehot, axis=1, keepdims=True) * mask)

    @pl.when(pid == 0)
    def _():
        acc_ref[0] = 0.0
        acc_ref[1] = 0.0

    acc_ref[0] += partO
    acc_ref[1] += part1

    @pl.when(pid == NPROG - 1)
    def _():
        loss_ref[...] = jnp.full((1, 1), -(acc_ref[0] + acc_ref[1]) / N,
                                 jnp.float32)

    m = logitsO[:, 0:1]
    idx = jnp.zeros((BR, 1), jnp.int32)
    for cc in range(1, NCLS):
        col = logitsO[:, cc:cc + 1]
        gt = col > m
        idx = jnp.where(gt, cc, idx)
        m = jnp.where(gt, col, m)
    pred_ref[...] = idx
    logits_ref[...] = logitsO


def _row_spec(cols):
    return pl.BlockSpec((BR, cols), lambda i: (i, 0))


def _full_spec(shape):
    nd = len(shape)
    return pl.BlockSpec(shape, lambda i, _n=nd: (0,) * nd)


def _lin1(xp, w1, b1, w4, b4):
    return pl.pallas_call(
        _lin1_body,
        grid=(NPROG,),
        in_specs=[
            _row_spec(NFEAT),
            _full_spec(w1.shape), _full_spec(b1.shape),
            _full_spec(w4.shape), _full_spec(b4.shape),
        ],
        out_specs=[_row_spec(NHID), _row_spec(NHID), _row_spec(1)],
        out_shape=[
            jax.ShapeDtypeStruct((NP, NHID), jnp.float32),
            jax.ShapeDtypeStruct((NP, NHID), jnp.float32),
            jax.ShapeDtypeStruct((NP, 1), jnp.float32),
        ],
    )(xp, w1, b1, w4, b4)


_BRS = 2048


def _scale(p0, p1, h1, h2):
    return pl.pallas_call(
        _scale_body,
        grid=(NP // _BRS,),
        in_specs=[
            pl.BlockSpec((_BRS, NHID), lambda i: (i, 0)),
            pl.BlockSpec((_BRS, NHID), lambda i: (i, 0)),
            pl.BlockSpec((_BRS, NHID), lambda i: (i, 0)),
            pl.BlockSpec((_BRS, NHID), lambda i: (i, 0)),
        ],
        out_specs=[
            pl.BlockSpec((_BRS, 1), lambda i: (i, 0)),
            pl.BlockSpec((_BRS, NHID), lambda i: (i, 0)),
            pl.BlockSpec((_BRS, NHID), lambda i: (i, 0)),
        ],
        out_shape=[
            jax.ShapeDtypeStruct((NP, 1), jnp.float32),
            jax.ShapeDtypeStruct((NP, NHID), jnp.float32),
            jax.ShapeDtypeStruct((NP, NHID), jnp.float32),
        ],
    )(p0, p1, h1, h2)


def _mid(a0, a1, hp1, hp2, dn, dinv, wa, ba, wb, bb):
    return pl.pallas_call(
        _mid_body,
        grid=(NPROG,),
        in_specs=[
            _row_spec(NHID), _row_spec(NHID),
            _row_spec(NHID), _row_spec(NHID),
            _row_spec(1), _row_spec(1),
            _full_spec(wa.shape), _full_spec(ba.shape),
            _full_spec(wb.shape), _full_spec(bb.shape),
        ],
        out_specs=[_row_spec(NHID)] * 6,
        out_shape=[jax.ShapeDtypeStruct((NP, NHID), jnp.float32)] * 6,
    )(a0, a1, hp1, hp2, dn, dinv, wa, ba, wb, bb)


def _final(a0, a1, h5, h6, dn, dinv, x1, x2, z1, z2, gf, y2, wlist):
    in_specs = [
        _row_spec(NHID), _row_spec(NHID),
        _row_spec(NHID), _row_spec(NHID),
        _row_spec(1), _row_spec(1),
        _row_spec(NHID), _row_spec(NHID),
        _row_spec(NHID), _row_spec(NHID),
        _row_spec(6), _row_spec(1),
    ] + [_full_spec(w.shape) for w in wlist]
    return pl.pallas_call(
        _final_body,
        grid=(NPROG,),
        in_specs=in_specs,
        out_specs=[
            pl.BlockSpec((1, 1), lambda i: (0, 0)),
            _row_spec(1),
            _row_spec(NCLS),
        ],
        out_shape=[
            jax.ShapeDtypeStruct((1, 1), jnp.float32),
            jax.ShapeDtypeStruct((NP, 1), jnp.int32),
            jax.ShapeDtypeStruct((NP, NCLS), jnp.float32),
        ],
        scratch_shapes=[pltpu.SMEM((2,), jnp.float32)],
    )(a0, a1, h5, h6, dn, dinv, x1, x2, z1, z2, gf, y2, *wlist)


def kernel(x_, edge_index, y, params):
    f32 = jnp.float32
    xp = jnp.pad(x_, ((0, NP - N), (0, 0)))
    gf = xp[:, NF + 1:]
    y2 = jnp.pad(y, (0, NP - N)).reshape(NP, 1)

    src = edge_index[0]
    dst = edge_index[1]
    epad = N + (jnp.arange(EPAD - E, dtype=jnp.int32) % (NP - N))
    src3 = jnp.concatenate([src, epad]).reshape(EROWS, CH)
    dst3 = jnp.concatenate([dst, epad]).reshape(EROWS, CH)
    edges3 = jnp.stack([src3, dst3], axis=1)
    zeros128 = jnp.zeros((NP, NHID), f32)

    def wb(name):
        W, b = params[name]
        return W, b.reshape(1, -1)

    w1, b1 = wb('conv1')
    w4, b4 = wb('conv4')

    _deg_kernel, _agg_kernel = _get_sc_kernels()
    ones128 = jnp.ones((CH, NHID), f32)
    deg_out = _deg_kernel(dst3, ones128, zeros128)
    p0, p1 = deg_out[:NP], deg_out[NP:]
    h1, h2, dinv = _lin1(xp, w1, b1, w4, b4)
    dn, h1s, h2s = _scale(p0, p1, h1, h2)

    w2, b2 = wb('conv2')
    w5, b5 = wb('conv5')
    a0, a1 = _agg_kernel(edges3, h1s, h2s, zeros128)
    x1, x2, h3, h4, h3s, h4s = _mid(a0, a1, h1, h2, dn, dinv, w2, b2, w5, b5)

    w3, b3 = wb('conv3')
    w6, b6 = wb('conv6')
    a0, a1 = _agg_kernel(edges3, h3s, h4s, zeros128)
    z1, z2, h5, h6, h5s, h6s = _mid(a0, a1, h3, h4, dn, dinv, w3, b3, w6, b6)

    a0, a1 = _agg_kernel(edges3, h5s, h6s, zeros128)

    wlist = []
    for name in ('fl1', 'fl2', 'fl3', 'fl4', 'fl5', 'fl6',
                 'co1', 'co2', 'co3', 'c11', 'c12', 'c13'):
        W, b = wb(name)
        wlist += [W, b]
    loss, pred, logits = _final(a0, a1, h5, h6, dn, dinv,
                                x1, x2, z1, z2, gf, y2, wlist)
    return loss.reshape(()), pred[:N, 0], logits[:N]
